# baseline (speedup 1.0000x reference)
import functools

import jax
import jax.numpy as jnp
from jax import lax
from jax.experimental import pallas as pl
from jax.experimental.pallas import tpu as pltpu
from jax.experimental.pallas import tpu_sc as plsc

N = 10000
DEG = 16
D = 256
L = 16
NW = 32
BASE = N // NW
EXTRA = N - BASE * NW


def _sc_body(ce_hbm, al_hbm, msg_hbm, out_hbm, msg_v, al_v, ce_v, out_v, sems):
    w = lax.axis_index("s") * 2 + lax.axis_index("c")
    count = BASE + jnp.where(w < EXTRA, 1, 0)
    start = w * BASE + jnp.minimum(w, EXTRA)

    def issue(i, slot):
        n = start + i
        pltpu.async_copy(msg_hbm.at[n], msg_v.at[slot], sems.at[slot])
        pltpu.async_copy(al_hbm.at[n], al_v.at[slot], sems.at[slot])
        pltpu.async_copy(ce_hbm.at[n, pl.ds(0, 1), :], ce_v.at[slot], sems.at[slot])

    def drain(i, slot):
        n = start + i
        pltpu.make_async_copy(msg_hbm.at[n], msg_v.at[slot], sems.at[slot]).wait()
        pltpu.make_async_copy(al_hbm.at[n], al_v.at[slot], sems.at[slot]).wait()
        pltpu.make_async_copy(ce_hbm.at[n, pl.ds(0, 1), :], ce_v.at[slot], sems.at[slot]).wait()

    issue(0, 0)

    def step(i, carry):
        slot = lax.rem(i, 2)

        @pl.when(i + 1 < count)
        def _():
            issue(i + 1, lax.rem(i + 1, 2))

        drain(i, slot)
        alv = al_v[slot, :]
        for j in range(D // L):
            acc = ce_v[slot, 0, pl.ds(j * L, L)]
            for d in range(DEG):
                acc = acc + alv[d] * msg_v[slot, d, pl.ds(j * L, L)]
            out_v[pl.ds(j * L, L)] = acc
        pltpu.sync_copy(out_v, out_hbm.at[start + i])
        return carry

    lax.fori_loop(0, count, step, 0)


def kernel(curr_emb, alpha, msg):
    al2 = jnp.squeeze(alpha, -1)
    mesh = plsc.VectorSubcoreMesh(core_axis_name="c", subcore_axis_name="s")
    k = functools.partial(
        pl.kernel,
        mesh=mesh,
        out_type=jax.ShapeDtypeStruct((N, D), jnp.float32),
        scratch_types=[
            pltpu.VMEM((2, DEG, D), jnp.float32),
            pltpu.VMEM((2, DEG), jnp.float32),
            pltpu.VMEM((2, 1, D), jnp.float32),
            pltpu.VMEM((D,), jnp.float32),
            pltpu.SemaphoreType.DMA((2,)),
        ],
    )(_sc_body)
    return k(curr_emb, al2, msg)


# speedup vs baseline: 2.3901x; 2.3901x over previous
import functools

import jax
import jax.numpy as jnp
from jax import lax
from jax.experimental import pallas as pl
from jax.experimental.pallas import tpu as pltpu
from jax.experimental.pallas import tpu_sc as plsc

N = 10000
DEG = 16
D = 256

N_SC = 2000
N_TC = N - N_SC
BN = 1000
G = N_TC // BN

L = 16
NW = 32
BASE = N_SC // NW
EXTRA = N_SC - BASE * NW


def _ce_copy(ce_hbm, ce_vmem, sems, block, slot):
    return pltpu.make_async_copy(
        ce_hbm.at[pl.ds(block * BN, BN), 0, :], ce_vmem.at[slot], sems.at[slot])


def _tc_body(ce_hbm, al_ref, msg_ref, out_ref, ce_vmem, sems):
    i = pl.program_id(0)
    slot = jax.lax.rem(i, 2)

    @pl.when(i == 0)
    def _():
        _ce_copy(ce_hbm, ce_vmem, sems, 0, 0).start()

    @pl.when(i + 1 < G)
    def _():
        _ce_copy(ce_hbm, ce_vmem, sems, i + 1, jax.lax.rem(i + 1, 2)).start()

    al = al_ref[...]
    m = msg_ref[...]
    acc = jnp.sum(al[:, :, None] * m, axis=1)
    _ce_copy(ce_hbm, ce_vmem, sems, i, slot).wait()
    out_ref[...] = acc + ce_vmem[slot]


def _tc_part(curr_emb, al2, msg):
    return pl.pallas_call(
        _tc_body,
        grid=(G,),
        in_specs=[
            pl.BlockSpec(memory_space=pl.ANY),
            pl.BlockSpec((BN, DEG), lambda i: (i, 0)),
            pl.BlockSpec((BN, DEG, D), lambda i: (i, 0, 0)),
        ],
        out_specs=pl.BlockSpec((BN, D), lambda i: (i, 0)),
        out_shape=jax.ShapeDtypeStruct((N_TC, D), jnp.float32),
        scratch_shapes=[
            pltpu.VMEM((2, BN, D), jnp.float32),
            pltpu.SemaphoreType.DMA((2,)),
        ],
    )(curr_emb, al2, msg)


def _sc_body(ce_hbm, al_hbm, msg_hbm, out_hbm, msg_v, al_v, ce_v, out_v, sems):
    w = lax.axis_index("s") * 2 + lax.axis_index("c")
    count = BASE + jnp.where(w < EXTRA, 1, 0)
    start = N_TC + w * BASE + jnp.minimum(w, EXTRA)

    def issue(i, slot):
        n = start + i
        pltpu.async_copy(msg_hbm.at[n], msg_v.at[slot], sems.at[slot])
        pltpu.async_copy(al_hbm.at[n], al_v.at[slot], sems.at[slot])
        pltpu.async_copy(ce_hbm.at[n, pl.ds(0, 1), :], ce_v.at[slot], sems.at[slot])

    def drain(i, slot):
        n = start + i
        pltpu.make_async_copy(msg_hbm.at[n], msg_v.at[slot], sems.at[slot]).wait()
        pltpu.make_async_copy(al_hbm.at[n], al_v.at[slot], sems.at[slot]).wait()
        pltpu.make_async_copy(ce_hbm.at[n, pl.ds(0, 1), :], ce_v.at[slot], sems.at[slot]).wait()

    issue(0, 0)

    def step(i, carry):
        slot = lax.rem(i, 2)

        @pl.when(i + 1 < count)
        def _():
            issue(i + 1, lax.rem(i + 1, 2))

        drain(i, slot)
        alv = al_v[slot, :]
        for j in range(D // L):
            acc = ce_v[slot, 0, pl.ds(j * L, L)]
            for d in range(DEG):
                acc = acc + alv[d] * msg_v[slot, d, pl.ds(j * L, L)]
            out_v[pl.ds(j * L, L)] = acc
        pltpu.sync_copy(out_v, out_hbm.at[start + i - N_TC])
        return carry

    lax.fori_loop(0, count, step, 0)


def _sc_part(curr_emb, al2, msg):
    mesh = plsc.VectorSubcoreMesh(core_axis_name="c", subcore_axis_name="s")
    k = functools.partial(
        pl.kernel,
        mesh=mesh,
        out_type=jax.ShapeDtypeStruct((N_SC, D), jnp.float32),
        scratch_types=[
            pltpu.VMEM((2, DEG, D), jnp.float32),
            pltpu.VMEM((2, DEG), jnp.float32),
            pltpu.VMEM((2, 1, D), jnp.float32),
            pltpu.VMEM((D,), jnp.float32),
            pltpu.SemaphoreType.DMA((2,)),
        ],
    )(_sc_body)
    return k(curr_emb, al2, msg)


def kernel(curr_emb, alpha, msg):
    al2 = jnp.squeeze(alpha, -1)
    sc_out = _sc_part(curr_emb, al2, msg)
    tc_out = _tc_part(curr_emb, al2, msg)
    return jnp.concatenate([tc_out, sc_out], axis=0)


# speedup vs baseline: 2.4268x; 1.0153x over previous
import functools

import jax
import jax.numpy as jnp
from jax import lax
from jax.experimental import pallas as pl
from jax.experimental.pallas import tpu as pltpu
from jax.experimental.pallas import tpu_sc as plsc

N = 10000
DEG = 16
D = 256

N_SC = 2000
N_TC = N - N_SC
BN = 1000
G = N_TC // BN

L = 16
NW = 32
BASE = N_SC // NW
EXTRA = N_SC - BASE * NW


def _ce_copy(ce_hbm, ce_vmem, sems, block, slot):
    return pltpu.make_async_copy(
        ce_hbm.at[pl.ds(block * BN, BN), 0, :], ce_vmem.at[slot], sems.at[slot])


def _tc_body(ce_hbm, al_ref, msg_ref, out_ref, ce_vmem, sems):
    i = pl.program_id(0)
    slot = jax.lax.rem(i, 2)

    @pl.when(i == 0)
    def _():
        _ce_copy(ce_hbm, ce_vmem, sems, 0, 0).start()

    @pl.when(i + 1 < G)
    def _():
        _ce_copy(ce_hbm, ce_vmem, sems, i + 1, jax.lax.rem(i + 1, 2)).start()

    al = al_ref[...]
    m = msg_ref[...]
    acc = jnp.sum(al[:, :, None] * m, axis=1)
    _ce_copy(ce_hbm, ce_vmem, sems, i, slot).wait()
    out_ref[...] = acc + ce_vmem[slot]


def _tc_part(curr_emb, al2, msg):
    return pl.pallas_call(
        _tc_body,
        grid=(G,),
        in_specs=[
            pl.BlockSpec(memory_space=pl.ANY),
            pl.BlockSpec((BN, DEG), lambda i: (i, 0)),
            pl.BlockSpec((BN, DEG, D), lambda i: (i, 0, 0)),
        ],
        out_specs=pl.BlockSpec((BN, D), lambda i: (i, 0)),
        out_shape=jax.ShapeDtypeStruct((N_TC, D), jnp.float32),
        scratch_shapes=[
            pltpu.VMEM((2, BN, D), jnp.float32),
            pltpu.SemaphoreType.DMA((2,)),
        ],
    )(curr_emb, al2, msg)


def _sc_body(ce_hbm, al_hbm, msg_hbm, out_hbm, msg_v, al_v, ce_v, out_v, sems, osems):
    w = lax.axis_index("s") * 2 + lax.axis_index("c")
    count = BASE + jnp.where(w < EXTRA, 1, 0)
    start = N_TC + w * BASE + jnp.minimum(w, EXTRA)

    def issue(i, slot):
        n = start + i
        pltpu.async_copy(msg_hbm.at[n], msg_v.at[slot], sems.at[slot])
        pltpu.async_copy(al_hbm.at[n], al_v.at[slot], sems.at[slot])
        pltpu.async_copy(ce_hbm.at[n, pl.ds(0, 1), :], ce_v.at[slot], sems.at[slot])

    def drain(i, slot):
        n = start + i
        pltpu.make_async_copy(msg_hbm.at[n], msg_v.at[slot], sems.at[slot]).wait()
        pltpu.make_async_copy(al_hbm.at[n], al_v.at[slot], sems.at[slot]).wait()
        pltpu.make_async_copy(ce_hbm.at[n, pl.ds(0, 1), :], ce_v.at[slot], sems.at[slot]).wait()

    issue(0, 0)

    def step(i, carry):
        slot = lax.rem(i, 2)

        @pl.when(i + 1 < count)
        def _():
            issue(i + 1, lax.rem(i + 1, 2))

        drain(i, slot)
        oslot = lax.rem(i, 4)

        @pl.when(i >= 4)
        def _():
            pltpu.make_async_copy(
                out_v.at[oslot], out_hbm.at[start + i - 4 - N_TC],
                osems.at[oslot]).wait()

        alv = al_v[slot, :]
        for j in range(D // L):
            acc = ce_v[slot, 0, pl.ds(j * L, L)]
            for d in range(DEG):
                acc = acc + alv[d] * msg_v[slot, d, pl.ds(j * L, L)]
            out_v[oslot, pl.ds(j * L, L)] = acc
        pltpu.async_copy(out_v.at[oslot], out_hbm.at[start + i - N_TC],
                         osems.at[oslot])
        return carry

    lax.fori_loop(0, count, step, 0)
    for k in range(4):
        i2 = count - 4 + k
        pltpu.make_async_copy(
            out_v.at[lax.rem(i2, 4)], out_hbm.at[start + i2 - N_TC],
            osems.at[lax.rem(i2, 4)]).wait()


def _sc_part(curr_emb, al2, msg):
    mesh = plsc.VectorSubcoreMesh(core_axis_name="c", subcore_axis_name="s")
    k = functools.partial(
        pl.kernel,
        mesh=mesh,
        out_type=jax.ShapeDtypeStruct((N_SC, D), jnp.float32),
        scratch_types=[
            pltpu.VMEM((2, DEG, D), jnp.float32),
            pltpu.VMEM((2, DEG), jnp.float32),
            pltpu.VMEM((2, 1, D), jnp.float32),
            pltpu.VMEM((4, D), jnp.float32),
            pltpu.SemaphoreType.DMA((2,)),
            pltpu.SemaphoreType.DMA((4,)),
        ],
    )(_sc_body)
    return k(curr_emb, al2, msg)


def kernel(curr_emb, alpha, msg):
    al2 = jnp.squeeze(alpha, -1)
    sc_out = _sc_part(curr_emb, al2, msg)
    tc_out = _tc_part(curr_emb, al2, msg)
    return jnp.concatenate([tc_out, sc_out], axis=0)


# speedup vs baseline: 2.4457x; 1.0078x over previous
import functools

import jax
import jax.numpy as jnp
from jax import lax
from jax.experimental import pallas as pl
from jax.experimental.pallas import tpu as pltpu
from jax.experimental.pallas import tpu_sc as plsc

N = 10000
DEG = 16
D = 256

N_SC = 1000
N_TC = N - N_SC
BN = 1000
G = N_TC // BN

L = 16
NW = 32
BASE = N_SC // NW
EXTRA = N_SC - BASE * NW


def _ce_copy(ce_hbm, ce_vmem, sems, block, slot):
    return pltpu.make_async_copy(
        ce_hbm.at[pl.ds(block * BN, BN), 0, :], ce_vmem.at[slot], sems.at[slot])


def _tc_body(ce_hbm, al_ref, msg_ref, out_ref, ce_vmem, sems):
    i = pl.program_id(0)
    slot = jax.lax.rem(i, 2)

    @pl.when(i == 0)
    def _():
        _ce_copy(ce_hbm, ce_vmem, sems, 0, 0).start()

    @pl.when(i + 1 < G)
    def _():
        _ce_copy(ce_hbm, ce_vmem, sems, i + 1, jax.lax.rem(i + 1, 2)).start()

    al = al_ref[...]
    m = msg_ref[...]
    acc = jnp.sum(al[:, :, None] * m, axis=1)
    _ce_copy(ce_hbm, ce_vmem, sems, i, slot).wait()
    out_ref[...] = acc + ce_vmem[slot]


def _tc_part(curr_emb, al2, msg):
    return pl.pallas_call(
        _tc_body,
        grid=(G,),
        in_specs=[
            pl.BlockSpec(memory_space=pl.ANY),
            pl.BlockSpec((BN, DEG), lambda i: (i, 0)),
            pl.BlockSpec((BN, DEG, D), lambda i: (i, 0, 0)),
        ],
        out_specs=pl.BlockSpec((BN, D), lambda i: (i, 0)),
        out_shape=jax.ShapeDtypeStruct((N_TC, D), jnp.float32),
        scratch_shapes=[
            pltpu.VMEM((2, BN, D), jnp.float32),
            pltpu.SemaphoreType.DMA((2,)),
        ],
    )(curr_emb, al2, msg)


def _sc_body(ce_hbm, al_hbm, msg_hbm, out_hbm, msg_v, al_v, ce_v, out_v, sems, osems):
    w = lax.axis_index("s") * 2 + lax.axis_index("c")
    count = BASE + jnp.where(w < EXTRA, 1, 0)
    start = N_TC + w * BASE + jnp.minimum(w, EXTRA)

    def issue(i, slot):
        n = start + i
        pltpu.async_copy(msg_hbm.at[n], msg_v.at[slot], sems.at[slot])
        pltpu.async_copy(al_hbm.at[n], al_v.at[slot], sems.at[slot])
        pltpu.async_copy(ce_hbm.at[n, pl.ds(0, 1), :], ce_v.at[slot], sems.at[slot])

    def drain(i, slot):
        n = start + i
        pltpu.make_async_copy(msg_hbm.at[n], msg_v.at[slot], sems.at[slot]).wait()
        pltpu.make_async_copy(al_hbm.at[n], al_v.at[slot], sems.at[slot]).wait()
        pltpu.make_async_copy(ce_hbm.at[n, pl.ds(0, 1), :], ce_v.at[slot], sems.at[slot]).wait()

    issue(0, 0)

    def step(i, carry):
        slot = lax.rem(i, 2)

        @pl.when(i + 1 < count)
        def _():
            issue(i + 1, lax.rem(i + 1, 2))

        drain(i, slot)
        oslot = lax.rem(i, 4)

        @pl.when(i >= 4)
        def _():
            pltpu.make_async_copy(
                out_v.at[oslot], out_hbm.at[start + i - 4 - N_TC],
                osems.at[oslot]).wait()

        alv = al_v[slot, :]
        for j in range(D // L):
            acc = ce_v[slot, 0, pl.ds(j * L, L)]
            for d in range(DEG):
                acc = acc + alv[d] * msg_v[slot, d, pl.ds(j * L, L)]
            out_v[oslot, pl.ds(j * L, L)] = acc
        pltpu.async_copy(out_v.at[oslot], out_hbm.at[start + i - N_TC],
                         osems.at[oslot])
        return carry

    lax.fori_loop(0, count, step, 0)
    for k in range(4):
        i2 = count - 4 + k
        pltpu.make_async_copy(
            out_v.at[lax.rem(i2, 4)], out_hbm.at[start + i2 - N_TC],
            osems.at[lax.rem(i2, 4)]).wait()


def _sc_part(curr_emb, al2, msg):
    mesh = plsc.VectorSubcoreMesh(core_axis_name="c", subcore_axis_name="s")
    k = functools.partial(
        pl.kernel,
        mesh=mesh,
        out_type=jax.ShapeDtypeStruct((N_SC, D), jnp.float32),
        scratch_types=[
            pltpu.VMEM((2, DEG, D), jnp.float32),
            pltpu.VMEM((2, DEG), jnp.float32),
            pltpu.VMEM((2, 1, D), jnp.float32),
            pltpu.VMEM((4, D), jnp.float32),
            pltpu.SemaphoreType.DMA((2,)),
            pltpu.SemaphoreType.DMA((4,)),
        ],
    )(_sc_body)
    return k(curr_emb, al2, msg)


def kernel(curr_emb, alpha, msg):
    al2 = jnp.squeeze(alpha, -1)
    sc_out = _sc_part(curr_emb, al2, msg)
    tc_out = _tc_part(curr_emb, al2, msg)
    return jnp.concatenate([tc_out, sc_out], axis=0)


# speedup vs baseline: 3.3327x; 1.3627x over previous
import jax
import jax.numpy as jnp
from jax.experimental import pallas as pl
from jax.experimental.pallas import tpu as pltpu

N = 10000
DEG = 16
D = 256
BN = 1000
G = N // BN


def _ce_copy(ce_hbm, ce_vmem, sems, block, slot):
    return pltpu.make_async_copy(
        ce_hbm.at[pl.ds(block * BN, BN), 0, :], ce_vmem.at[slot], sems.at[slot])


def _body(ce_hbm, al_ref, msg_ref, out_ref, ce_vmem, sems):
    i = pl.program_id(0)
    slot = jax.lax.rem(i, 2)

    @pl.when(i == 0)
    def _():
        _ce_copy(ce_hbm, ce_vmem, sems, 0, 0).start()

    @pl.when(i + 1 < G)
    def _():
        _ce_copy(ce_hbm, ce_vmem, sems, i + 1, jax.lax.rem(i + 1, 2)).start()

    al = al_ref[...]
    m = msg_ref[...]
    acc = jnp.sum(al[:, :, None] * m, axis=1)
    _ce_copy(ce_hbm, ce_vmem, sems, i, slot).wait()
    out_ref[...] = acc + ce_vmem[slot]


def kernel(curr_emb, alpha, msg):
    al2 = jnp.squeeze(alpha, -1)
    return pl.pallas_call(
        _body,
        grid=(G,),
        in_specs=[
            pl.BlockSpec(memory_space=pl.ANY),
            pl.BlockSpec((BN, DEG), lambda i: (i, 0)),
            pl.BlockSpec((BN, DEG, D), lambda i: (i, 0, 0)),
        ],
        out_specs=pl.BlockSpec((BN, D), lambda i: (i, 0)),
        out_shape=jax.ShapeDtypeStruct((N, D), jnp.float32),
        scratch_shapes=[
            pltpu.VMEM((2, BN, D), jnp.float32),
            pltpu.SemaphoreType.DMA((2,)),
        ],
    )(curr_emb, al2, msg)
